# baseline (speedup 1.0000x reference)
import functools

import jax
import jax.numpy as jnp
from jax import lax
from jax.experimental import pallas as pl
from jax.experimental.pallas import tpu as pltpu
from jax.experimental.pallas import tpu_sc as plsc

D = 64
EPS = 1e-12
L = 16
NC, NS = 2, 16
NW = NC * NS
SUB = 128
CHUNK = 512
NSUB = CHUNK // SUB


def _rsqrt(v):
    i = lax.bitcast_convert_type(v, jnp.int32)
    i = jnp.int32(0x5F3759DF) - (i >> 1)
    y = lax.bitcast_convert_type(i, jnp.float32)
    y = y * (1.5 - 0.5 * v * y * y)
    y = y * (1.5 - 0.5 * v * y * y)
    y = y * (1.5 - 0.5 * v * y * y)
    return y


def _ln_group_body(rows_v, pbuf, qbuf, cmv, gmv, btv, g):
    base = g * L
    nj = D // L
    for i in range(L):
        xc = [rows_v[base + i, pl.ds(j * L, L)] + cmv[j] for j in range(nj)]
        for j in range(nj):
            rows_v[base + i, pl.ds(j * L, L)] = xc[j]
        p = (xc[0] + xc[1]) + (xc[2] + xc[3])
        q = (xc[0] * xc[0] + xc[1] * xc[1]) + (xc[2] * xc[2] + xc[3] * xc[3])
        pbuf[pl.ds(i * L, L)] = p
        qbuf[pl.ds(i * L, L)] = q
    rowsel = lax.iota(jnp.int32, L) * L
    t0 = plsc.load_gather(pbuf, [rowsel]) + plsc.load_gather(pbuf, [rowsel + 1])
    q0 = plsc.load_gather(qbuf, [rowsel]) + plsc.load_gather(qbuf, [rowsel + 1])
    for c in range(2, L, 2):
        t0 = t0 + (plsc.load_gather(pbuf, [rowsel + c])
                   + plsc.load_gather(pbuf, [rowsel + c + 1]))
        q0 = q0 + (plsc.load_gather(qbuf, [rowsel + c])
                   + plsc.load_gather(qbuf, [rowsel + c + 1]))
    mean = t0 * (1.0 / D)
    var = q0 * (1.0 / D) - mean * mean
    rinv = _rsqrt(var + EPS)
    for i in range(L):
        m_i = mean[i]
        r_i = rinv[i]
        for j in range(nj):
            xc = rows_v[base + i, pl.ds(j * L, L)]
            rows_v[base + i, pl.ds(j * L, L)] = (xc - m_i) * (gmv[j] * r_i) + btv[j]


def _make_sc_kernel(n_rows):
    rows_per_w = n_rows // NW
    n_chunks = rows_per_w // CHUNK
    mesh = plsc.VectorSubcoreMesh(core_axis_name="c", subcore_axis_name="s")

    @functools.partial(
        pl.kernel,
        mesh=mesh,
        out_type=jax.ShapeDtypeStruct((n_rows, D), jnp.float32),
        compiler_params=pltpu.CompilerParams(
            needs_layout_passes=False, use_tc_tiling_on_sc=False
        ),
        scratch_types=[
            pltpu.VMEM((rows_per_w // SUB, SUB), jnp.int32),
            pltpu.VMEM((CHUNK, D), jnp.float32),
            pltpu.VMEM((3, D), jnp.float32),
            pltpu.VMEM((L * L,), jnp.float32),
            pltpu.VMEM((L * L,), jnp.float32),
            pltpu.SemaphoreType.DMA,
        ],
    )
    def sc_kernel(ids_hbm, table_hbm, prm_hbm, out_hbm, idx_v, rows_v, prm_v,
                  pbuf, qbuf, sem):
        wid = lax.axis_index("s") * NC + lax.axis_index("c")
        idx_rows = rows_per_w // SUB
        pltpu.sync_copy(prm_hbm, prm_v)
        pltpu.sync_copy(ids_hbm.at[pl.ds(wid * idx_rows, idx_rows)], idx_v)
        cmv = [prm_v[0, pl.ds(j * L, L)] for j in range(D // L)]
        gmv = [prm_v[1, pl.ds(j * L, L)] for j in range(D // L)]
        btv = [prm_v[2, pl.ds(j * L, L)] for j in range(D // L)]

        def chunk_body(c, _):
            row_base = wid * rows_per_w + c * CHUNK
            copies = [
                pltpu.async_copy(
                    table_hbm.at[idx_v.at[c * NSUB + j]],
                    rows_v.at[pl.ds(j * SUB, SUB)],
                    sem,
                )
                for j in range(NSUB)
            ]
            for cp in copies:
                cp.wait()
            lax.fori_loop(
                0, CHUNK // L,
                lambda g, carry: (
                    _ln_group_body(rows_v, pbuf, qbuf, cmv, gmv, btv, g),
                    carry,
                )[1],
                0,
            )
            pltpu.sync_copy(rows_v, out_hbm.at[pl.ds(row_base, CHUNK)])
            return 0

        lax.fori_loop(0, n_chunks, chunk_body, 0)

    return sc_kernel


def kernel(input_ids, table, common, gamma, beta):
    b, s = input_ids.shape
    n_rows = b * s
    ids = input_ids.reshape(n_rows).astype(jnp.int32).reshape(n_rows // SUB, SUB)
    prm = jnp.concatenate(
        [common.reshape(1, D), gamma.reshape(1, D), beta.reshape(1, D)], axis=0
    )
    out = _make_sc_kernel(n_rows)(ids, table, prm)
    return out.reshape(b, s, D)


# speedup vs baseline: 1.0541x; 1.0541x over previous
import functools

import jax
import jax.numpy as jnp
from jax import lax
from jax.experimental import pallas as pl
from jax.experimental.pallas import tpu as pltpu
from jax.experimental.pallas import tpu_sc as plsc

D = 64
EPS = 1e-12
L = 16
NC, NS = 2, 16
NW = NC * NS
SUB = 128
CHUNK = 512
NSUB = CHUNK // SUB


def _rsqrt(v):
    i = lax.bitcast_convert_type(v, jnp.int32)
    i = jnp.int32(0x5F3759DF) - (i >> 1)
    y = lax.bitcast_convert_type(i, jnp.float32)
    y = y * (1.5 - 0.5 * v * y * y)
    y = y * (1.5 - 0.5 * v * y * y)
    y = y * (1.5 - 0.5 * v * y * y)
    return y


def _ln_group_body(rows_v, pbuf, qbuf, cmv, gmv, btv, g):
    base = g * L
    nj = D // L
    for i in range(L):
        xc = [rows_v[base + i, pl.ds(j * L, L)] + cmv[j] for j in range(nj)]
        for j in range(nj):
            rows_v[base + i, pl.ds(j * L, L)] = xc[j]
        p = (xc[0] + xc[1]) + (xc[2] + xc[3])
        q = (xc[0] * xc[0] + xc[1] * xc[1]) + (xc[2] * xc[2] + xc[3] * xc[3])
        pbuf[pl.ds(i * L, L)] = p
        qbuf[pl.ds(i * L, L)] = q
    rowsel = lax.iota(jnp.int32, L) * L
    t0 = plsc.load_gather(pbuf, [rowsel]) + plsc.load_gather(pbuf, [rowsel + 1])
    q0 = plsc.load_gather(qbuf, [rowsel]) + plsc.load_gather(qbuf, [rowsel + 1])
    for c in range(2, L, 2):
        t0 = t0 + (plsc.load_gather(pbuf, [rowsel + c])
                   + plsc.load_gather(pbuf, [rowsel + c + 1]))
        q0 = q0 + (plsc.load_gather(qbuf, [rowsel + c])
                   + plsc.load_gather(qbuf, [rowsel + c + 1]))
    mean = t0 * (1.0 / D)
    var = q0 * (1.0 / D) - mean * mean
    rinv = _rsqrt(var + EPS)
    for i in range(L):
        m_i = mean[i]
        r_i = rinv[i]
        for j in range(nj):
            xc = rows_v[base + i, pl.ds(j * L, L)]
            rows_v[base + i, pl.ds(j * L, L)] = (xc - m_i) * (gmv[j] * r_i) + btv[j]


def _make_sc_kernel(n_rows):
    rows_per_w = n_rows // NW
    n_chunks = rows_per_w // CHUNK
    mesh = plsc.VectorSubcoreMesh(core_axis_name="c", subcore_axis_name="s")

    @functools.partial(
        pl.kernel,
        mesh=mesh,
        out_type=jax.ShapeDtypeStruct((n_rows, D), jnp.float32),
        compiler_params=pltpu.CompilerParams(
            needs_layout_passes=False, use_tc_tiling_on_sc=False
        ),
        scratch_types=[
            pltpu.VMEM((rows_per_w // SUB, SUB), jnp.int32),
            pltpu.VMEM((CHUNK, D), jnp.float32),
            pltpu.VMEM((CHUNK, D), jnp.float32),
            pltpu.VMEM((3, D), jnp.float32),
            pltpu.VMEM((L * L,), jnp.float32),
            pltpu.VMEM((L * L,), jnp.float32),
            pltpu.SemaphoreType.DMA,
            pltpu.SemaphoreType.DMA,
            pltpu.SemaphoreType.DMA,
            pltpu.SemaphoreType.DMA,
        ],
    )
    def sc_kernel(ids_hbm, table_hbm, prm_hbm, out_hbm, idx_v, rows_a, rows_b,
                  prm_v, pbuf, qbuf, sga, sgb, soa, sob):
        wid = lax.axis_index("s") * NC + lax.axis_index("c")
        idx_rows = rows_per_w // SUB
        pltpu.sync_copy(prm_hbm, prm_v)
        pltpu.sync_copy(ids_hbm.at[pl.ds(wid * idx_rows, idx_rows)], idx_v)
        cmv = [prm_v[0, pl.ds(j * L, L)] for j in range(D // L)]
        gmv = [prm_v[1, pl.ds(j * L, L)] for j in range(D // L)]
        btv = [prm_v[2, pl.ds(j * L, L)] for j in range(D // L)]

        def fire_gathers(c, buf, sem):
            for j in range(NSUB):
                pltpu.async_copy(
                    table_hbm.at[idx_v.at[c * NSUB + j]],
                    buf.at[pl.ds(j * SUB, SUB)],
                    sem,
                )

        def wait_gathers(buf, sem):
            for j in range(NSUB):
                pltpu.make_async_copy(
                    table_hbm.at[idx_v.at[j]], buf.at[pl.ds(j * SUB, SUB)], sem
                ).wait()

        def compute(buf):
            lax.fori_loop(
                0, CHUNK // L,
                lambda g, carry: (
                    _ln_group_body(buf, pbuf, qbuf, cmv, gmv, btv, g),
                    carry,
                )[1],
                0,
            )

        def issue_out(c, buf, sem):
            row_base = wid * rows_per_w + c * CHUNK
            pltpu.async_copy(buf, out_hbm.at[pl.ds(row_base, CHUNK)], sem)

        def wait_out(buf, sem):
            pltpu.make_async_copy(buf, out_hbm.at[pl.ds(0, CHUNK)], sem).wait()

        fire_gathers(0, rows_a, sga)

        def pair_body(p, _):
            ca = 2 * p
            cb = 2 * p + 1

            @pl.when(p > 0)
            def _():
                wait_out(rows_b, sob)

            fire_gathers(cb, rows_b, sgb)
            wait_gathers(rows_a, sga)
            compute(rows_a)
            issue_out(ca, rows_a, soa)

            wait_out(rows_a, soa)
            fire_gathers(jnp.minimum(cb + 1, n_chunks - 1), rows_a, sga)
            wait_gathers(rows_b, sgb)
            compute(rows_b)
            issue_out(cb, rows_b, sob)
            return 0

        lax.fori_loop(0, n_chunks // 2, pair_body, 0)
        wait_gathers(rows_a, sga)
        wait_out(rows_b, sob)

    return sc_kernel


def kernel(input_ids, table, common, gamma, beta):
    b, s = input_ids.shape
    n_rows = b * s
    ids = input_ids.reshape(n_rows).astype(jnp.int32).reshape(n_rows // SUB, SUB)
    prm = jnp.concatenate(
        [common.reshape(1, D), gamma.reshape(1, D), beta.reshape(1, D)], axis=0
    )
    out = _make_sc_kernel(n_rows)(ids, table, prm)
    return out.reshape(b, s, D)
